# baseline (speedup 1.0000x reference)
import functools

import jax
import jax.numpy as jnp
from jax import lax
from jax.experimental import pallas as pl
from jax.experimental.pallas import tpu as pltpu
from jax.experimental.pallas import tpu_sc as plsc

_N = 10000
_E = 320000
_D = 128

_NC = 2
_NS = 16
_NW = _NC * _NS
_CH = 80
_NBUF = 3
_GLA = _NBUF - 1
_NCH0 = 123
_NCH1 = 127
_EPW0 = _NCH0 * _CH
_EPW1 = _NCH1 * _CH
_EP = _NS * (_EPW0 + _EPW1)
_SLAB = 624
_TAIL = _N - _NS * _SLAB


def _mm_body(x_ref, w_ref, o_ref):
    o_ref[...] = jnp.dot(x_ref[...], w_ref[...],
                         preferred_element_type=jnp.float32)


def _matmul(x, W):
    return pl.pallas_call(
        _mm_body,
        grid=(10,),
        in_specs=[
            pl.BlockSpec((1000, _D), lambda i: (i, 0)),
            pl.BlockSpec((_D, _D), lambda i: (0, 0)),
        ],
        out_specs=pl.BlockSpec((1000, _D), lambda i: (i, 0)),
        out_shape=jax.ShapeDtypeStruct((_N, _D), jnp.float32),
    )(x, W)


def _add_body(a_ref, b_ref, o_ref):
    o_ref[...] = a_ref[...] + b_ref[...]


def _combine(p0, p1):
    return pl.pallas_call(
        _add_body,
        grid=(10,),
        in_specs=[
            pl.BlockSpec((1000, _D), lambda i: (i, 0)),
            pl.BlockSpec((1000, _D), lambda i: (i, 0)),
        ],
        out_specs=pl.BlockSpec((1000, _D), lambda i: (i, 0)),
        out_shape=jax.ShapeDtypeStruct((_N, _D), jnp.float32),
    )(p0, p1)


_mesh = plsc.VectorSubcoreMesh(core_axis_name="c", subcore_axis_name="s")


@functools.partial(
    pl.kernel,
    mesh=_mesh,
    out_type=jax.ShapeDtypeStruct((_NC, _N, _D), jnp.float32),
    scratch_types=[
        *([pltpu.VMEM((_CH,), jnp.int32)] * _NBUF),
        *([pltpu.VMEM((_CH,), jnp.int32)] * _NBUF),
        *([pltpu.VMEM((_CH,), jnp.float32)] * _NBUF),
        *([pltpu.VMEM((_CH, _D), jnp.float32)] * _NBUF),
        pltpu.VMEM_SHARED((_N, _D), jnp.float32),
        *([pltpu.SemaphoreType.DMA] * _NBUF),
        *([pltpu.SemaphoreType.DMA] * _NBUF),
    ],
)
def _sc_spmm(sup_hbm, src_hbm, dst_hbm, ev_hbm, out_hbm, *scr):
    c = lax.axis_index("c")
    s = lax.axis_index("s")
    nchunk = jnp.where(c == 0, _NCH0, _NCH1)
    ebase = c * _NS * _EPW0 + s * jnp.where(c == 0, _EPW0, _EPW1)
    srcv = scr[0:_NBUF]
    dstv = scr[_NBUF:2 * _NBUF]
    evv = scr[2 * _NBUF:3 * _NBUF]
    rows = scr[3 * _NBUF:4 * _NBUF]
    acc = scr[4 * _NBUF]
    gsem = scr[4 * _NBUF + 1:5 * _NBUF + 1]
    isem = scr[5 * _NBUF + 1:6 * _NBUF + 1]
    rows0 = rows[0]

    def idx_start(k, b):
        base = ebase + k * _CH
        pltpu.async_copy(src_hbm.at[pl.ds(base, _CH)], srcv[b], isem[b])
        pltpu.async_copy(dst_hbm.at[pl.ds(base, _CH)], dstv[b], isem[b])
        pltpu.async_copy(ev_hbm.at[pl.ds(base, _CH)], evv[b], isem[b])

    def idx_wait(k, b):
        base = ebase + k * _CH
        pltpu.make_async_copy(src_hbm.at[pl.ds(base, _CH)], srcv[b],
                              isem[b]).wait()
        pltpu.make_async_copy(dst_hbm.at[pl.ds(base, _CH)], dstv[b],
                              isem[b]).wait()
        pltpu.make_async_copy(ev_hbm.at[pl.ds(base, _CH)], evv[b],
                              isem[b]).wait()

    def zb_body(i, carry):
        for b in range(_D // 16):
            rows0[i, pl.ds(b * 16, 16)] = jnp.zeros((16,), jnp.float32)
        return carry

    lax.fori_loop(0, _CH, zb_body, 0)
    for kz in range(_SLAB // _CH):
        pltpu.sync_copy(rows0, acc.at[pl.ds(s * _SLAB + kz * _CH, _CH)])
    pltpu.sync_copy(rows0.at[pl.ds(0, _SLAB - (_SLAB // _CH) * _CH)],
                    acc.at[pl.ds(s * _SLAB + (_SLAB // _CH) * _CH,
                                 _SLAB - (_SLAB // _CH) * _CH)])

    @pl.when(s == _NS - 1)
    def _zero_tail():
        pltpu.sync_copy(rows0.at[pl.ds(0, _TAIL)],
                        acc.at[pl.ds(_NS * _SLAB, _TAIL)])

    plsc.subcore_barrier()

    for p in range(_GLA):
        idx_start(p, p)
        idx_wait(p, p)
        pltpu.async_copy(sup_hbm.at[srcv[p]], rows[p], gsem[p])
    idx_start(_GLA, _GLA)

    def scale_and_scatter(b):
        def mul_body(g, inner):
            evg = evv[b][pl.ds(g * 16, 16)]
            for j in range(16):
                v = evg[j]
                e = g * 16 + j
                for blk in range(_D // 16):
                    sl = rows[b][e, pl.ds(blk * 16, 16)]
                    rows[b][e, pl.ds(blk * 16, 16)] = sl * v
            return inner

        lax.fori_loop(0, _CH // 16, mul_body, 0)
        pltpu.sync_copy(rows[b], acc.at[dstv[b]], add=True)

    def grp_body(kk, carry):
        for b in range(_NBUF):
            k = kk * _NBUF + b
            bg = (b + _GLA) % _NBUF
            pltpu.make_async_copy(sup_hbm.at[srcv[b]], rows[b],
                                  gsem[b]).wait()

            @pl.when(k + _GLA < nchunk)
            def _next_gather():
                idx_wait(k + _GLA, bg)
                pltpu.async_copy(sup_hbm.at[srcv[bg]], rows[bg], gsem[bg])

            scale_and_scatter(b)

            @pl.when(k + _NBUF < nchunk)
            def _next_idx():
                idx_start(k + _NBUF, b)

        return carry

    lax.fori_loop(0, nchunk // _NBUF, grp_body, 0)

    @pl.when(c == 1)
    def _epilogue():
        pltpu.make_async_copy(sup_hbm.at[srcv[(_NCH1 - 1) % _NBUF]],
                              rows[(_NCH1 - 1) % _NBUF],
                              gsem[(_NCH1 - 1) % _NBUF]).wait()
        scale_and_scatter((_NCH1 - 1) % _NBUF)

    plsc.subcore_barrier()

    pltpu.sync_copy(acc.at[pl.ds(s * _SLAB, _SLAB)],
                    out_hbm.at[c, pl.ds(s * _SLAB, _SLAB)])

    @pl.when(s == _NS - 1)
    def _copy_tail():
        pltpu.sync_copy(acc.at[pl.ds(_NS * _SLAB, _TAIL)],
                        out_hbm.at[c, pl.ds(_NS * _SLAB, _TAIL)])


def kernel(x, edge_index, edge_vals, W):
    support = _matmul(x, W)
    dst = edge_index[0]
    src = edge_index[1]
    partials = _sc_spmm(support, src, dst, edge_vals)
    return _combine(partials[0], partials[1])


# speedup vs baseline: 1.0020x; 1.0020x over previous
import functools

import jax
import jax.numpy as jnp
from jax import lax
from jax.experimental import pallas as pl
from jax.experimental.pallas import tpu as pltpu
from jax.experimental.pallas import tpu_sc as plsc

_N = 10000
_E = 320000
_D = 128

_NC = 2
_NS = 16
_NW = _NC * _NS
_CH = 80
_NBUF = 3
_GLA = _NBUF - 1
_NCH0 = 126
_NCH1 = 124
_EPW0 = _NCH0 * _CH
_EPW1 = _NCH1 * _CH
_EP = _NS * (_EPW0 + _EPW1)
_SLAB = 624
_TAIL = _N - _NS * _SLAB


def _mm_body(x_ref, w_ref, o_ref):
    o_ref[...] = jnp.dot(x_ref[...], w_ref[...],
                         preferred_element_type=jnp.float32)


def _matmul(x, W):
    return pl.pallas_call(
        _mm_body,
        grid=(10,),
        in_specs=[
            pl.BlockSpec((1000, _D), lambda i: (i, 0)),
            pl.BlockSpec((_D, _D), lambda i: (0, 0)),
        ],
        out_specs=pl.BlockSpec((1000, _D), lambda i: (i, 0)),
        out_shape=jax.ShapeDtypeStruct((_N, _D), jnp.float32),
    )(x, W)


def _add_body(a_ref, b_ref, o_ref):
    o_ref[...] = a_ref[...] + b_ref[...]


def _combine(p0, p1):
    return pl.pallas_call(
        _add_body,
        grid=(10,),
        in_specs=[
            pl.BlockSpec((1000, _D), lambda i: (i, 0)),
            pl.BlockSpec((1000, _D), lambda i: (i, 0)),
        ],
        out_specs=pl.BlockSpec((1000, _D), lambda i: (i, 0)),
        out_shape=jax.ShapeDtypeStruct((_N, _D), jnp.float32),
    )(p0, p1)


_mesh = plsc.VectorSubcoreMesh(core_axis_name="c", subcore_axis_name="s")


@functools.partial(
    pl.kernel,
    mesh=_mesh,
    out_type=jax.ShapeDtypeStruct((_NC, _N, _D), jnp.float32),
    scratch_types=[
        *([pltpu.VMEM((_CH,), jnp.int32)] * _NBUF),
        *([pltpu.VMEM((_CH,), jnp.int32)] * _NBUF),
        *([pltpu.VMEM((_CH,), jnp.float32)] * _NBUF),
        *([pltpu.VMEM((_CH, _D), jnp.float32)] * _NBUF),
        pltpu.VMEM_SHARED((_N, _D), jnp.float32),
        *([pltpu.SemaphoreType.DMA] * _NBUF),
        *([pltpu.SemaphoreType.DMA] * _NBUF),
    ],
)
def _sc_spmm(sup_hbm, src_hbm, dst_hbm, ev_hbm, out_hbm, *scr):
    c = lax.axis_index("c")
    s = lax.axis_index("s")
    nchunk = jnp.where(c == 0, _NCH0, _NCH1)
    ebase = c * _NS * _EPW0 + s * jnp.where(c == 0, _EPW0, _EPW1)
    srcv = scr[0:_NBUF]
    dstv = scr[_NBUF:2 * _NBUF]
    evv = scr[2 * _NBUF:3 * _NBUF]
    rows = scr[3 * _NBUF:4 * _NBUF]
    acc = scr[4 * _NBUF]
    gsem = scr[4 * _NBUF + 1:5 * _NBUF + 1]
    isem = scr[5 * _NBUF + 1:6 * _NBUF + 1]
    rows0 = rows[0]

    def idx_start(k, b):
        base = ebase + k * _CH
        pltpu.async_copy(src_hbm.at[pl.ds(base, _CH)], srcv[b], isem[b])
        pltpu.async_copy(dst_hbm.at[pl.ds(base, _CH)], dstv[b], isem[b])
        pltpu.async_copy(ev_hbm.at[pl.ds(base, _CH)], evv[b], isem[b])

    def idx_wait(k, b):
        base = ebase + k * _CH
        pltpu.make_async_copy(src_hbm.at[pl.ds(base, _CH)], srcv[b],
                              isem[b]).wait()
        pltpu.make_async_copy(dst_hbm.at[pl.ds(base, _CH)], dstv[b],
                              isem[b]).wait()
        pltpu.make_async_copy(ev_hbm.at[pl.ds(base, _CH)], evv[b],
                              isem[b]).wait()

    def zb_body(i, carry):
        for b in range(_D // 16):
            rows0[i, pl.ds(b * 16, 16)] = jnp.zeros((16,), jnp.float32)
        return carry

    lax.fori_loop(0, _CH, zb_body, 0)
    for kz in range(_SLAB // _CH):
        pltpu.sync_copy(rows0, acc.at[pl.ds(s * _SLAB + kz * _CH, _CH)])
    pltpu.sync_copy(rows0.at[pl.ds(0, _SLAB - (_SLAB // _CH) * _CH)],
                    acc.at[pl.ds(s * _SLAB + (_SLAB // _CH) * _CH,
                                 _SLAB - (_SLAB // _CH) * _CH)])

    @pl.when(s == _NS - 1)
    def _zero_tail():
        pltpu.sync_copy(rows0.at[pl.ds(0, _TAIL)],
                        acc.at[pl.ds(_NS * _SLAB, _TAIL)])

    plsc.subcore_barrier()

    for p in range(_GLA):
        idx_start(p, p)
        idx_wait(p, p)
        pltpu.async_copy(sup_hbm.at[srcv[p]], rows[p], gsem[p])
    idx_start(_GLA, _GLA)

    def scale_and_scatter(b):
        def mul_body(g, inner):
            evg = evv[b][pl.ds(g * 16, 16)]
            for j in range(16):
                v = evg[j]
                e = g * 16 + j
                for blk in range(_D // 16):
                    sl = rows[b][e, pl.ds(blk * 16, 16)]
                    rows[b][e, pl.ds(blk * 16, 16)] = sl * v
            return inner

        lax.fori_loop(0, _CH // 16, mul_body, 0)
        pltpu.sync_copy(rows[b], acc.at[dstv[b]], add=True)

    def grp_body(kk, carry):
        for b in range(_NBUF):
            k = kk * _NBUF + b
            bg = (b + _GLA) % _NBUF
            pltpu.make_async_copy(sup_hbm.at[srcv[b]], rows[b],
                                  gsem[b]).wait()

            @pl.when(k + _GLA < nchunk)
            def _next_gather():
                idx_wait(k + _GLA, bg)
                pltpu.async_copy(sup_hbm.at[srcv[bg]], rows[bg], gsem[bg])

            scale_and_scatter(b)

            @pl.when(k + _NBUF < nchunk)
            def _next_idx():
                idx_start(k + _NBUF, b)

        return carry

    lax.fori_loop(0, nchunk // _NBUF, grp_body, 0)

    @pl.when(c == 1)
    def _epilogue():
        pltpu.make_async_copy(sup_hbm.at[srcv[(_NCH1 - 1) % _NBUF]],
                              rows[(_NCH1 - 1) % _NBUF],
                              gsem[(_NCH1 - 1) % _NBUF]).wait()
        scale_and_scatter((_NCH1 - 1) % _NBUF)

    plsc.subcore_barrier()

    pltpu.sync_copy(acc.at[pl.ds(s * _SLAB, _SLAB)],
                    out_hbm.at[c, pl.ds(s * _SLAB, _SLAB)])

    @pl.when(s == _NS - 1)
    def _copy_tail():
        pltpu.sync_copy(acc.at[pl.ds(_NS * _SLAB, _TAIL)],
                        out_hbm.at[c, pl.ds(_NS * _SLAB, _TAIL)])


def kernel(x, edge_index, edge_vals, W):
    support = _matmul(x, W)
    dst = edge_index[0]
    src = edge_index[1]
    partials = _sc_spmm(support, src, dst, edge_vals)
    return _combine(partials[0], partials[1])


# speedup vs baseline: 1.0030x; 1.0010x over previous
import functools

import jax
import jax.numpy as jnp
from jax import lax
from jax.experimental import pallas as pl
from jax.experimental.pallas import tpu as pltpu
from jax.experimental.pallas import tpu_sc as plsc

_N = 10000
_E = 320000
_D = 128

_NC = 2
_NS = 16
_NW = _NC * _NS
_CH = 80
_NBUF = 3
_GLA = _NBUF - 1
_NCH0 = 126
_NCH1 = 124
_EPW0 = _NCH0 * _CH
_EPW1 = _NCH1 * _CH
_EP = _NS * (_EPW0 + _EPW1)
_SLAB = 624
_TAIL = _N - _NS * _SLAB


def _mm_body(x_ref, w_ref, o_ref):
    o_ref[...] = jnp.dot(x_ref[...], w_ref[...],
                         preferred_element_type=jnp.float32)


def _matmul(x, W):
    return pl.pallas_call(
        _mm_body,
        grid=(10,),
        in_specs=[
            pl.BlockSpec((1000, _D), lambda i: (i, 0)),
            pl.BlockSpec((_D, _D), lambda i: (0, 0)),
        ],
        out_specs=pl.BlockSpec((1000, _D), lambda i: (i, 0)),
        out_shape=jax.ShapeDtypeStruct((_N, _D), jnp.float32),
    )(x, W)


def _add_body(a_ref, b_ref, o_ref):
    o_ref[...] = a_ref[...] + b_ref[...]


def _combine(p0, p1):
    return pl.pallas_call(
        _add_body,
        grid=(10,),
        in_specs=[
            pl.BlockSpec((1000, _D), lambda i: (i, 0)),
            pl.BlockSpec((1000, _D), lambda i: (i, 0)),
        ],
        out_specs=pl.BlockSpec((1000, _D), lambda i: (i, 0)),
        out_shape=jax.ShapeDtypeStruct((_N, _D), jnp.float32),
    )(p0, p1)


_mesh = plsc.VectorSubcoreMesh(core_axis_name="c", subcore_axis_name="s")


@functools.partial(
    pl.kernel,
    mesh=_mesh,
    out_type=jax.ShapeDtypeStruct((_NC, _N, _D), jnp.float32),
    scratch_types=[
        *([pltpu.VMEM((_CH,), jnp.int32)] * _NBUF),
        *([pltpu.VMEM((_CH,), jnp.int32)] * _NBUF),
        *([pltpu.VMEM((_CH,), jnp.float32)] * _NBUF),
        *([pltpu.VMEM((_CH, _D), jnp.float32)] * _NBUF),
        pltpu.VMEM_SHARED((_N, _D), jnp.float32),
        *([pltpu.SemaphoreType.DMA] * _NBUF),
        *([pltpu.SemaphoreType.DMA] * _NBUF),
    ],
)
def _sc_spmm(sup_hbm, src_hbm, dst_hbm, ev_hbm, out_hbm, *scr):
    c = lax.axis_index("c")
    s = lax.axis_index("s")
    nchunk = jnp.where(c == 0, _NCH0, _NCH1)
    ebase = c * _NS * _EPW0 + s * jnp.where(c == 0, _EPW0, _EPW1)
    srcv = scr[0:_NBUF]
    dstv = scr[_NBUF:2 * _NBUF]
    evv = scr[2 * _NBUF:3 * _NBUF]
    rows = scr[3 * _NBUF:4 * _NBUF]
    acc = scr[4 * _NBUF]
    gsem = scr[4 * _NBUF + 1:5 * _NBUF + 1]
    isem = scr[5 * _NBUF + 1:6 * _NBUF + 1]
    rows0 = rows[0]

    def idx_start(k, b):
        base = ebase + k * _CH
        pltpu.async_copy(src_hbm.at[pl.ds(base, _CH)], srcv[b], isem[b])
        pltpu.async_copy(dst_hbm.at[pl.ds(base, _CH)], dstv[b], isem[b])
        pltpu.async_copy(ev_hbm.at[pl.ds(base, _CH)], evv[b], isem[b])

    def idx_wait(k, b):
        base = ebase + k * _CH
        pltpu.make_async_copy(src_hbm.at[pl.ds(base, _CH)], srcv[b],
                              isem[b]).wait()
        pltpu.make_async_copy(dst_hbm.at[pl.ds(base, _CH)], dstv[b],
                              isem[b]).wait()
        pltpu.make_async_copy(ev_hbm.at[pl.ds(base, _CH)], evv[b],
                              isem[b]).wait()

    for p in range(_NBUF):
        idx_start(p, p)

    def zb_body(i, carry):
        for b in range(_D // 16):
            rows0[i, pl.ds(b * 16, 16)] = jnp.zeros((16,), jnp.float32)
        return carry

    lax.fori_loop(0, _CH, zb_body, 0)
    for kz in range(_SLAB // _CH):
        pltpu.sync_copy(rows0, acc.at[pl.ds(s * _SLAB + kz * _CH, _CH)])
    pltpu.sync_copy(rows0.at[pl.ds(0, _SLAB - (_SLAB // _CH) * _CH)],
                    acc.at[pl.ds(s * _SLAB + (_SLAB // _CH) * _CH,
                                 _SLAB - (_SLAB // _CH) * _CH)])

    @pl.when(s == _NS - 1)
    def _zero_tail():
        pltpu.sync_copy(rows0.at[pl.ds(0, _TAIL)],
                        acc.at[pl.ds(_NS * _SLAB, _TAIL)])

    for p in range(_GLA):
        idx_wait(p, p)
        pltpu.async_copy(sup_hbm.at[srcv[p]], rows[p], gsem[p])
    plsc.subcore_barrier()

    def scale_and_scatter(b):
        def mul_body(g, inner):
            evg = evv[b][pl.ds(g * 16, 16)]
            for j in range(16):
                v = evg[j]
                e = g * 16 + j
                for blk in range(_D // 16):
                    sl = rows[b][e, pl.ds(blk * 16, 16)]
                    rows[b][e, pl.ds(blk * 16, 16)] = sl * v
            return inner

        lax.fori_loop(0, _CH // 16, mul_body, 0)
        pltpu.sync_copy(rows[b], acc.at[dstv[b]], add=True)

    def grp_body(kk, carry):
        for b in range(_NBUF):
            k = kk * _NBUF + b
            bg = (b + _GLA) % _NBUF
            pltpu.make_async_copy(sup_hbm.at[srcv[b]], rows[b],
                                  gsem[b]).wait()

            @pl.when(k + _GLA < nchunk)
            def _next_gather():
                idx_wait(k + _GLA, bg)
                pltpu.async_copy(sup_hbm.at[srcv[bg]], rows[bg], gsem[bg])

            scale_and_scatter(b)

            @pl.when(k + _NBUF < nchunk)
            def _next_idx():
                idx_start(k + _NBUF, b)

        return carry

    lax.fori_loop(0, nchunk // _NBUF, grp_body, 0)

    @pl.when(c == 1)
    def _epilogue():
        pltpu.make_async_copy(sup_hbm.at[srcv[(_NCH1 - 1) % _NBUF]],
                              rows[(_NCH1 - 1) % _NBUF],
                              gsem[(_NCH1 - 1) % _NBUF]).wait()
        scale_and_scatter((_NCH1 - 1) % _NBUF)

    plsc.subcore_barrier()

    pltpu.sync_copy(acc.at[pl.ds(s * _SLAB, _SLAB)],
                    out_hbm.at[c, pl.ds(s * _SLAB, _SLAB)])

    @pl.when(s == _NS - 1)
    def _copy_tail():
        pltpu.sync_copy(acc.at[pl.ds(_NS * _SLAB, _TAIL)],
                        out_hbm.at[c, pl.ds(_NS * _SLAB, _TAIL)])


def kernel(x, edge_index, edge_vals, W):
    support = _matmul(x, W)
    dst = edge_index[0]
    src = edge_index[1]
    partials = _sc_spmm(support, src, dst, edge_vals)
    return _combine(partials[0], partials[1])
